# baseline (speedup 1.0000x reference)
import functools

import jax
import jax.numpy as jnp
from jax import lax
from jax.experimental import pallas as pl
from jax.experimental.pallas import tpu as pltpu
from jax.experimental.pallas import tpu_sc as plsc

N, D = 50000, 128
NC, NS = 2, 16
NW = NC * NS
CHUNK = 1568
R = 112
GROUPS = R // 16
NBLOCKS = CHUNK // R
PAIRS = NBLOCKS // 2

_GATHER_DNUMS = lax.GatherDimensionNumbers(
    offset_dims=(), collapsed_slice_dims=(0,), start_index_map=(0,))


def _perm(x, idx):
    return lax.gather(
        x, idx.reshape(16, 1), _GATHER_DNUMS, (1,),
        mode=lax.GatherScatterMode.PROMISE_IN_BOUNDS)


def _body(gu_hbm, gi_hbm, out_hbm, ua, ia, ub, ib, obuf,
          sem_ua, sem_ia, sem_ub, sem_ib):
    wid = lax.axis_index("s") * NC + lax.axis_index("c")
    base_w = jnp.minimum(wid * CHUNK, N - CHUNK)

    lanes = lax.iota(jnp.int32, 16)
    xor_idx = {h: lanes ^ h for h in (8, 4, 2, 1)}
    bitrev = (
        ((lanes & 1) << 3) | (((lanes >> 1) & 1) << 2)
        | (((lanes >> 2) & 1) << 1) | ((lanes >> 3) & 1))

    def combine(a, b, h):
        sel = (lanes & h) == 0
        return jnp.where(sel, a + _perm(a, xor_idx[h]), b + _perm(b, xor_idx[h]))

    def start(base, u_ref, i_ref, su, si):
        pltpu.async_copy(gu_hbm.at[pl.ds(base, R), :], u_ref, su)
        pltpu.async_copy(gi_hbm.at[pl.ds(base, R), :], i_ref, si)

    def wait(u_ref, i_ref, su, si):
        pltpu.make_async_copy(gu_hbm.at[pl.ds(0, R), :], u_ref, su).wait()
        pltpu.make_async_copy(gi_hbm.at[pl.ds(0, R), :], i_ref, si).wait()

    def compute(u_ref, i_ref, off):
        @plsc.parallel_loop(0, GROUPS, unroll=2)
        def group(g):
            r0 = g * 16
            vs = []
            for rr in range(16):
                r = r0 + rr
                ps = [u_ref[r, pl.ds(k * 16, 16)] * i_ref[r, pl.ds(k * 16, 16)]
                      for k in range(8)]
                while len(ps) > 1:
                    ps = [ps[i] + ps[i + 1] for i in range(0, len(ps), 2)]
                vs.append(ps[0])
            acc = vs
            for h in (8, 4, 2, 1):
                acc = [combine(acc[i], acc[i + 1], h)
                       for i in range(0, len(acc), 2)]
            obuf[pl.ds(off + r0, 16)] = _perm(acc[0], bitrev)

    start(base_w, ua, ia, sem_ua, sem_ia)

    def pair(p, _):
        b0 = 2 * p
        start(base_w + (b0 + 1) * R, ub, ib, sem_ub, sem_ib)
        wait(ua, ia, sem_ua, sem_ia)
        compute(ua, ia, b0 * R)

        @pl.when(p < PAIRS - 1)
        def _():
            start(base_w + (b0 + 2) * R, ua, ia, sem_ua, sem_ia)

        wait(ub, ib, sem_ub, sem_ib)
        compute(ub, ib, (b0 + 1) * R)
        return 0

    lax.fori_loop(0, PAIRS, pair, 0)
    pltpu.sync_copy(obuf, out_hbm.at[pl.ds(base_w, CHUNK)])


@jax.jit
def kernel(gu, gi):
    f = functools.partial(
        pl.kernel,
        mesh=plsc.VectorSubcoreMesh(core_axis_name="c", subcore_axis_name="s"),
        out_type=jax.ShapeDtypeStruct((N,), jnp.float32),
        scratch_types=[
            pltpu.VMEM((R, D), jnp.float32),
            pltpu.VMEM((R, D), jnp.float32),
            pltpu.VMEM((R, D), jnp.float32),
            pltpu.VMEM((R, D), jnp.float32),
            pltpu.VMEM((CHUNK,), jnp.float32),
            pltpu.SemaphoreType.DMA,
            pltpu.SemaphoreType.DMA,
            pltpu.SemaphoreType.DMA,
            pltpu.SemaphoreType.DMA,
        ],
    )(_body)
    return f(gu, gi)


# speedup vs baseline: 1.1596x; 1.1596x over previous
import functools

import jax
import jax.numpy as jnp
from jax import lax
from jax.experimental import pallas as pl
from jax.experimental.pallas import tpu as pltpu
from jax.experimental.pallas import tpu_sc as plsc

N, D = 50000, 128
NC, NS = 2, 16
NW = NC * NS
CHUNK = 1568
R = 112
GROUPS = R // 16
NBLOCKS = CHUNK // R
PAIRS = NBLOCKS // 2

_GATHER_DNUMS = lax.GatherDimensionNumbers(
    offset_dims=(), collapsed_slice_dims=(0,), start_index_map=(0,))


def _perm(x, idx):
    return lax.gather(
        x, idx.reshape(16, 1), _GATHER_DNUMS, (1,),
        mode=lax.GatherScatterMode.PROMISE_IN_BOUNDS)


def _body(gu_hbm, gi_hbm, out_hbm, ua, ia, ub, ib, obuf,
          sem_ua, sem_ia, sem_ub, sem_ib):
    wid = lax.axis_index("s") * NC + lax.axis_index("c")
    base_w = jnp.minimum(wid * CHUNK, N - CHUNK)

    lanes = lax.iota(jnp.int32, 16)
    xor_idx = {h: lanes ^ h for h in (8, 4, 2, 1)}
    bitrev = (
        ((lanes & 1) << 3) | (((lanes >> 1) & 1) << 2)
        | (((lanes >> 2) & 1) << 1) | ((lanes >> 3) & 1))

    def combine(a, b, h):
        sel = (lanes & h) == 0
        return jnp.where(sel, a + _perm(a, xor_idx[h]), b + _perm(b, xor_idx[h]))

    def start(base, u_ref, i_ref, su, si):
        pltpu.async_copy(gu_hbm.at[pl.ds(base, R), :], u_ref, su)
        pltpu.async_copy(gi_hbm.at[pl.ds(base, R), :], i_ref, si)

    def wait(u_ref, i_ref, su, si):
        pltpu.make_async_copy(gu_hbm.at[pl.ds(0, R), :], u_ref, su).wait()
        pltpu.make_async_copy(gi_hbm.at[pl.ds(0, R), :], i_ref, si).wait()

    hs = (8, 4, 2, 1)

    def compute(u_ref, i_ref, off):
        def group(g, _):
            r0 = g * 16
            stack = []
            for rr in range(16):
                r = r0 + rr
                ps = [u_ref[r, pl.ds(k * 16, 16)] * i_ref[r, pl.ds(k * 16, 16)]
                      for k in range(8)]
                while len(ps) > 1:
                    ps = [ps[i] + ps[i + 1] for i in range(0, len(ps), 2)]
                node = (0, ps[0])
                while stack and stack[-1][0] == node[0]:
                    lvl, prev = stack.pop()
                    node = (lvl + 1, combine(prev, node[1], hs[lvl]))
                stack.append(node)
            obuf[pl.ds(off + r0, 16)] = _perm(stack[0][1], bitrev)
            return 0

        lax.fori_loop(0, GROUPS, group, 0)

    start(base_w, ua, ia, sem_ua, sem_ia)

    def pair(p, _):
        b0 = 2 * p
        start(base_w + (b0 + 1) * R, ub, ib, sem_ub, sem_ib)
        wait(ua, ia, sem_ua, sem_ia)
        compute(ua, ia, b0 * R)

        @pl.when(p < PAIRS - 1)
        def _():
            start(base_w + (b0 + 2) * R, ua, ia, sem_ua, sem_ia)

        wait(ub, ib, sem_ub, sem_ib)
        compute(ub, ib, (b0 + 1) * R)
        return 0

    lax.fori_loop(0, PAIRS, pair, 0)
    pltpu.sync_copy(obuf, out_hbm.at[pl.ds(base_w, CHUNK)])


@jax.jit
def kernel(gu, gi):
    f = functools.partial(
        pl.kernel,
        mesh=plsc.VectorSubcoreMesh(core_axis_name="c", subcore_axis_name="s"),
        out_type=jax.ShapeDtypeStruct((N,), jnp.float32),
        scratch_types=[
            pltpu.VMEM((R, D), jnp.float32),
            pltpu.VMEM((R, D), jnp.float32),
            pltpu.VMEM((R, D), jnp.float32),
            pltpu.VMEM((R, D), jnp.float32),
            pltpu.VMEM((CHUNK,), jnp.float32),
            pltpu.SemaphoreType.DMA,
            pltpu.SemaphoreType.DMA,
            pltpu.SemaphoreType.DMA,
            pltpu.SemaphoreType.DMA,
        ],
    )(_body)
    return f(gu, gi)


# speedup vs baseline: 1.7339x; 1.4953x over previous
import functools

import jax
import jax.numpy as jnp
from jax import lax
from jax.experimental import pallas as pl
from jax.experimental.pallas import tpu as pltpu
from jax.experimental.pallas import tpu_sc as plsc

N, D = 50000, 128
NC, NS = 2, 16
NW = NC * NS

BTC = 2048
N_TC = 18 * BTC

R = 112
GROUPS = R // 16
NBLOCKS = 4
PAIRS = NBLOCKS // 2
CHUNK = NBLOCKS * R

_GATHER_DNUMS = lax.GatherDimensionNumbers(
    offset_dims=(), collapsed_slice_dims=(0,), start_index_map=(0,))


def _perm(x, idx):
    return lax.gather(
        x, idx.reshape(16, 1), _GATHER_DNUMS, (1,),
        mode=lax.GatherScatterMode.PROMISE_IN_BOUNDS)


def _sc_body(gu_hbm, gi_hbm, out_hbm, ua, ia, ub, ib, obuf,
             sem_ua, sem_ia, sem_ub, sem_ib):
    wid = lax.axis_index("s") * NC + lax.axis_index("c")
    base_w = jnp.minimum(N_TC + wid * CHUNK, N - CHUNK)

    lanes = lax.iota(jnp.int32, 16)
    xor_idx = {h: lanes ^ h for h in (8, 4, 2, 1)}
    bitrev = (
        ((lanes & 1) << 3) | (((lanes >> 1) & 1) << 2)
        | (((lanes >> 2) & 1) << 1) | ((lanes >> 3) & 1))

    def combine(a, b, h):
        sel = (lanes & h) == 0
        return jnp.where(sel, a + _perm(a, xor_idx[h]), b + _perm(b, xor_idx[h]))

    def start(base, u_ref, i_ref, su, si):
        pltpu.async_copy(gu_hbm.at[pl.ds(base, R), :], u_ref, su)
        pltpu.async_copy(gi_hbm.at[pl.ds(base, R), :], i_ref, si)

    def wait(u_ref, i_ref, su, si):
        pltpu.make_async_copy(gu_hbm.at[pl.ds(0, R), :], u_ref, su).wait()
        pltpu.make_async_copy(gi_hbm.at[pl.ds(0, R), :], i_ref, si).wait()

    hs = (8, 4, 2, 1)

    def compute(u_ref, i_ref, off):
        def group(g, _):
            r0 = g * 16
            stack = []
            for rr in range(16):
                r = r0 + rr
                ps = [u_ref[r, pl.ds(k * 16, 16)] * i_ref[r, pl.ds(k * 16, 16)]
                      for k in range(8)]
                while len(ps) > 1:
                    ps = [ps[i] + ps[i + 1] for i in range(0, len(ps), 2)]
                node = (0, ps[0])
                while stack and stack[-1][0] == node[0]:
                    lvl, prev = stack.pop()
                    node = (lvl + 1, combine(prev, node[1], hs[lvl]))
                stack.append(node)
            obuf[pl.ds(off + r0, 16)] = _perm(stack[0][1], bitrev)
            return 0

        lax.fori_loop(0, GROUPS, group, 0)

    start(base_w, ua, ia, sem_ua, sem_ia)

    def pair(p, _):
        b0 = 2 * p
        start(base_w + (b0 + 1) * R, ub, ib, sem_ub, sem_ib)
        wait(ua, ia, sem_ua, sem_ia)
        compute(ua, ia, b0 * R)

        @pl.when(p < PAIRS - 1)
        def _():
            start(base_w + (b0 + 2) * R, ua, ia, sem_ua, sem_ia)

        wait(ub, ib, sem_ub, sem_ib)
        compute(ub, ib, (b0 + 1) * R)
        return 0

    lax.fori_loop(0, PAIRS, pair, 0)
    pltpu.sync_copy(obuf, out_hbm.at[pl.ds(base_w - N_TC, CHUNK)])


def _sc_call(gu, gi):
    f = functools.partial(
        pl.kernel,
        mesh=plsc.VectorSubcoreMesh(core_axis_name="c", subcore_axis_name="s"),
        out_type=jax.ShapeDtypeStruct((N - N_TC,), jnp.float32),
        scratch_types=[
            pltpu.VMEM((R, D), jnp.float32),
            pltpu.VMEM((R, D), jnp.float32),
            pltpu.VMEM((R, D), jnp.float32),
            pltpu.VMEM((R, D), jnp.float32),
            pltpu.VMEM((CHUNK,), jnp.float32),
            pltpu.SemaphoreType.DMA,
            pltpu.SemaphoreType.DMA,
            pltpu.SemaphoreType.DMA,
            pltpu.SemaphoreType.DMA,
        ],
    )(_sc_body)
    return f(gu, gi)


def _tc_body(u_ref, i_ref, o_ref):
    o_ref[...] = jnp.sum(u_ref[...] * i_ref[...], axis=1)


def _tc_call(gu, gi):
    return pl.pallas_call(
        _tc_body,
        grid=(N_TC // BTC,),
        in_specs=[
            pl.BlockSpec((BTC, D), lambda b: (b, 0)),
            pl.BlockSpec((BTC, D), lambda b: (b, 0)),
        ],
        out_specs=pl.BlockSpec((BTC,), lambda b: (b,)),
        out_shape=jax.ShapeDtypeStruct((N_TC,), jnp.float32),
    )(gu, gi)


@jax.jit
def kernel(gu, gi):
    out_sc = _sc_call(gu, gi)
    out_tc = _tc_call(gu, gi)
    return jnp.concatenate([out_tc, out_sc])


# speedup vs baseline: 1.7416x; 1.0044x over previous
import jax
import jax.numpy as jnp
from jax import lax
from jax.experimental import pallas as pl

N, D = 50000, 128
BTC = 2048
GRID = (N + BTC - 1) // BTC


def _tc_body(u_ref, i_ref, o_ref):
    prod = u_ref[...] * i_ref[...]
    ones = jnp.ones((D, 1), jnp.float32)
    o_ref[...] = lax.dot_general(
        prod, ones, (((1,), (0,)), ((), ())),
        preferred_element_type=jnp.float32)


@jax.jit
def kernel(gu, gi):
    out = pl.pallas_call(
        _tc_body,
        grid=(GRID,),
        in_specs=[
            pl.BlockSpec((BTC, D), lambda b: (b, 0)),
            pl.BlockSpec((BTC, D), lambda b: (b, 0)),
        ],
        out_specs=pl.BlockSpec((BTC, 1), lambda b: (b, 0)),
        out_shape=jax.ShapeDtypeStruct((N, 1), jnp.float32),
    )(gu, gi)
    return out[:, 0]


# speedup vs baseline: 2.0638x; 1.1850x over previous
import jax
import jax.numpy as jnp
from jax import lax
from jax.experimental import pallas as pl

N, D = 50000, 128
BTC = 1024
GRID = (N + BTC - 1) // BTC


def _tc_body(u_ref, i_ref, o_ref):
    prod = u_ref[...] * i_ref[...]
    ones = jnp.ones((1, D), jnp.float32)
    o_ref[...] = lax.dot_general(
        ones, prod, (((1,), (1,)), ((), ())),
        preferred_element_type=jnp.float32)[None]


@jax.jit
def kernel(gu, gi):
    out = pl.pallas_call(
        _tc_body,
        grid=(GRID,),
        in_specs=[
            pl.BlockSpec((BTC, D), lambda b: (b, 0)),
            pl.BlockSpec((BTC, D), lambda b: (b, 0)),
        ],
        out_specs=pl.BlockSpec((1, 1, BTC), lambda b: (b, 0, 0)),
        out_shape=jax.ShapeDtypeStruct((GRID, 1, BTC), jnp.float32),
    )(gu, gi)
    return out.reshape(-1)[:N]


# speedup vs baseline: 3.9239x; 1.9013x over previous
import jax
import jax.numpy as jnp
from jax import lax
from jax.experimental import pallas as pl
from jax.experimental.pallas import tpu as pltpu

N, D = 50000, 128
BR = 1000
NB = N // BR
NBUF = 5
NT = NB // NBUF


def _tc_body(gu_hbm, gi_hbm, o_ref, ubufs, ibufs, sem_u, sem_i):
    ones = jnp.ones((1, D), jnp.float32)

    def start(b, u):
        pltpu.async_copy(gu_hbm.at[pl.ds(b * BR, BR), :], ubufs.at[u], sem_u.at[u])
        pltpu.async_copy(gi_hbm.at[pl.ds(b * BR, BR), :], ibufs.at[u], sem_i.at[u])

    def wait(u):
        pltpu.make_async_copy(gu_hbm.at[pl.ds(0, BR), :], ubufs.at[u], sem_u.at[u]).wait()
        pltpu.make_async_copy(gi_hbm.at[pl.ds(0, BR), :], ibufs.at[u], sem_i.at[u]).wait()

    for u in range(NBUF):
        start(u, u)

    def outer(t, _):
        for u in range(NBUF):
            b = t * NBUF + u
            wait(u)
            prod = ubufs[u] * ibufs[u]
            o_ref[pl.ds(t * NBUF + u, 1), :] = lax.dot_general(
                ones, prod, (((1,), (1,)), ((), ())),
                preferred_element_type=jnp.float32)

            @pl.when(t < NT - 1)
            def _():
                start(b + NBUF, u)
        return 0

    lax.fori_loop(0, NT, outer, 0)


@jax.jit
def kernel(gu, gi):
    out = pl.pallas_call(
        _tc_body,
        in_specs=[
            pl.BlockSpec(memory_space=pltpu.MemorySpace.HBM),
            pl.BlockSpec(memory_space=pltpu.MemorySpace.HBM),
        ],
        out_shape=jax.ShapeDtypeStruct((NB, BR), jnp.float32),
        scratch_shapes=[
            pltpu.VMEM((NBUF, BR, D), jnp.float32),
            pltpu.VMEM((NBUF, BR, D), jnp.float32),
            pltpu.SemaphoreType.DMA((NBUF,)),
            pltpu.SemaphoreType.DMA((NBUF,)),
        ],
    )(gu, gi)
    return out.reshape(N)
